# baseline (speedup 1.0000x reference)
import functools

import jax
import jax.numpy as jnp
from jax import lax
from jax.experimental import pallas as pl
from jax.experimental.pallas import tpu as pltpu
from jax.experimental.pallas import tpu_sc as plsc

LR = 0.001
ANTI = 0.4
PRECISION_FLOOR = 1e-30

B = 16384
D = 128
O = 1024

HB = B // 2
BT = 1024
GRID_H = HB // BT

NW = 32
RPW = HB // NW
CH = 128
NCH = RPW // CH
ZROWS = O // 16


def _matmul_top2_body(x_ref, w_ref, y_ref, a1_ref, a2_ref, eye_ref):
    i = pl.program_id(0)

    @pl.when(i == 0)
    def _build_eye():
        r = lax.broadcasted_iota(jnp.int32, (BT, BT), 0)
        c = lax.broadcasted_iota(jnp.int32, (BT, BT), 1)
        eye_ref[...] = (r == c).astype(jnp.float32)

    x = x_ref[...]
    w = w_ref[...]
    t = lax.dot_general(x, w, (((1,), (1,)), ((), ())),
                        preferred_element_type=jnp.float32)
    y_ref[...] = t
    iota_f = lax.broadcasted_iota(jnp.int32, (BT, O), 1).astype(jnp.float32)
    v1 = jnp.max(t, axis=1, keepdims=True)
    a1f = jnp.max(jnp.where(t == v1, iota_f, -1.0), axis=1, keepdims=True)
    t2 = jnp.where(iota_f == a1f, -jnp.inf, t)
    v2 = jnp.max(t2, axis=1, keepdims=True)
    a2f = jnp.max(jnp.where(t2 == v2, iota_f, -1.0), axis=1, keepdims=True)
    hi1 = jnp.floor(a1f * (1.0 / 32.0))
    lo1 = a1f - hi1 * 32.0
    hi2 = jnp.floor(a2f * (1.0 / 32.0))
    lo2 = a2f - hi2 * 32.0
    ab = jnp.concatenate([hi1, lo1, hi2, lo2], axis=1)
    abr = lax.dot_general(ab, eye_ref[...], (((0,), (0,)), ((), ())),
                          preferred_element_type=jnp.float32)
    rows = BT // CH
    a1_ref[pl.ds(i * rows, rows)] = (
        abr[0:1] * 32.0 + abr[1:2]).astype(jnp.int32).reshape(rows, CH)
    a2_ref[pl.ds(i * rows, rows)] = (
        abr[2:3] * 32.0 + abr[3:4]).astype(jnp.int32).reshape(rows, CH)


def _matmul_top2_body_aliased(y_in_ref, x_ref, w_ref, y_ref, a1_ref, a2_ref,
                              eye_ref):
    del y_in_ref
    _matmul_top2_body(x_ref, w_ref, y_ref, a1_ref, a2_ref, eye_ref)


_IDX_OUT = [
    jax.ShapeDtypeStruct((HB // CH, CH), jnp.int32),
    jax.ShapeDtypeStruct((HB // CH, CH), jnp.int32),
]

_matmul_top2_h0 = pl.pallas_call(
    _matmul_top2_body,
    grid=(GRID_H,),
    in_specs=[
        pl.BlockSpec((BT, D), lambda i: (i, 0)),
        pl.BlockSpec((O, D), lambda i: (0, 0)),
    ],
    out_specs=[
        pl.BlockSpec((BT, O), lambda i: (i, 0)),
        pl.BlockSpec((HB // CH, CH), lambda i: (0, 0)),
        pl.BlockSpec((HB // CH, CH), lambda i: (0, 0)),
    ],
    out_shape=[jax.ShapeDtypeStruct((B, O), jnp.float32)] + _IDX_OUT,
    scratch_shapes=[pltpu.VMEM((BT, BT), jnp.float32)],
)

_matmul_top2_h1 = pl.pallas_call(
    _matmul_top2_body_aliased,
    grid=(GRID_H,),
    in_specs=[
        pl.BlockSpec(memory_space=pl.ANY),
        pl.BlockSpec((BT, D), lambda i: (i + GRID_H, 0)),
        pl.BlockSpec((O, D), lambda i: (0, 0)),
    ],
    out_specs=[
        pl.BlockSpec((BT, O), lambda i: (i + GRID_H, 0)),
        pl.BlockSpec((HB // CH, CH), lambda i: (0, 0)),
        pl.BlockSpec((HB // CH, CH), lambda i: (0, 0)),
    ],
    out_shape=[jax.ShapeDtypeStruct((B, O), jnp.float32)] + _IDX_OUT,
    scratch_shapes=[pltpu.VMEM((BT, BT), jnp.float32)],
    input_output_aliases={0: 0},
)


def _make_scatter_body(half):
    def _scatter_body(x_hbm, a1_hbm, a2_hbm, out1_hbm, out2_hbm,
                      idx1_v, idx2_v, xrows_v, zero_v, acc1_sh, acc2_sh,
                      xsems):
        cid = lax.axis_index("c")
        sid = lax.axis_index("s")
        wid = cid * 16 + sid
        base = half * HB + wid * RPW

        xcopies = [
            pltpu.async_copy(x_hbm.at[pl.ds(base + j * CH, CH)],
                             xrows_v.at[pl.ds(j * CH, CH)], xsems.at[j])
            for j in range(NCH)
        ]

        pltpu.sync_copy(a1_hbm.at[pl.ds(wid * NCH, NCH)], idx1_v)
        pltpu.sync_copy(a2_hbm.at[pl.ds(wid * NCH, NCH)], idx2_v)

        def _zero_row(i, carry):
            for j in range(D // 16):
                zero_v[i, pl.ds(j * 16, 16)] = jnp.zeros((16,), jnp.float32)
            return carry
        lax.fori_loop(0, ZROWS, _zero_row, 0)

        pltpu.sync_copy(zero_v, acc1_sh.at[pl.ds(sid * ZROWS, ZROWS)])
        pltpu.sync_copy(zero_v, acc2_sh.at[pl.ds(sid * ZROWS, ZROWS)])

        plsc.subcore_barrier()

        for j in range(NCH):
            xcopies[j].wait()
            pltpu.sync_copy(xrows_v.at[pl.ds(j * CH, CH)],
                            acc1_sh.at[idx1_v.at[j]], add=True)
            pltpu.sync_copy(xrows_v.at[pl.ds(j * CH, CH)],
                            acc2_sh.at[idx2_v.at[j]], add=True)

        plsc.subcore_barrier()

        pltpu.sync_copy(acc1_sh.at[pl.ds(sid * ZROWS, ZROWS)],
                        out1_hbm.at[cid, pl.ds(sid * ZROWS, ZROWS)])
        pltpu.sync_copy(acc2_sh.at[pl.ds(sid * ZROWS, ZROWS)],
                        out2_hbm.at[cid, pl.ds(sid * ZROWS, ZROWS)])

    return _scatter_body


@functools.cache
def _scatter_accumulate(half):
    return functools.partial(
        pl.kernel,
        out_type=[
            jax.ShapeDtypeStruct((2, O, D), jnp.float32),
            jax.ShapeDtypeStruct((2, O, D), jnp.float32),
        ],
        mesh=plsc.VectorSubcoreMesh(core_axis_name="c", subcore_axis_name="s"),
        scratch_types=[
            pltpu.VMEM((NCH, CH), jnp.int32),
            pltpu.VMEM((NCH, CH), jnp.int32),
            pltpu.VMEM((RPW, D), jnp.float32),
            pltpu.VMEM((ZROWS, D), jnp.float32),
            pltpu.VMEM_SHARED((O, D), jnp.float32),
            pltpu.VMEM_SHARED((O, D), jnp.float32),
            pltpu.SemaphoreType.DMA((NCH,)),
        ],
    )(_make_scatter_body(half))


def _finalize_body(w_ref, acc1a_ref, acc2a_ref, acc1b_ref, acc2b_ref,
                   neww_ref):
    w = w_ref[...]
    s1 = acc1a_ref[0] + acc1a_ref[1] + acc1b_ref[0] + acc1b_ref[1]
    s2 = acc2a_ref[0] + acc2a_ref[1] + acc2b_ref[0] + acc2b_ref[1]
    dsb = s1 - ANTI * s2
    xx = jnp.sum(w * dsb, axis=1, keepdims=True)
    ds = dsb - xx * w
    nc = jnp.maximum(jnp.max(jnp.abs(ds)), PRECISION_FLOOR)
    neww_ref[...] = w + LR * (ds / nc)


_finalize = pl.pallas_call(
    _finalize_body,
    out_shape=jax.ShapeDtypeStruct((O, D), jnp.float32),
)


def kernel(input, W):
    y1, a1a, a2a = _matmul_top2_h0(input, W)
    acc1a, acc2a = _scatter_accumulate(0)(input, a1a, a2a)
    y, a1b, a2b = _matmul_top2_h1(y1, input, W)
    acc1b, acc2b = _scatter_accumulate(1)(input, a1b, a2b)
    new_W = _finalize(W, acc1a, acc2a, acc1b, acc2b)
    return (y, new_W)


# speedup vs baseline: 1.1311x; 1.1311x over previous
import functools

import jax
import jax.numpy as jnp
from jax import lax
from jax.experimental import pallas as pl
from jax.experimental.pallas import tpu as pltpu
from jax.experimental.pallas import tpu_sc as plsc

LR = 0.001
ANTI = 0.4
PRECISION_FLOOR = 1e-30

B = 16384
D = 128
O = 1024

BT = 1024
GRID = B // BT

NW = 32
ROWS_PER_W = B // NW
CH = 128
NCH = ROWS_PER_W // CH
ZROWS = O // 16


def _matmul_top2_body(x_ref, w_ref, y_ref, a1_ref, a2_ref, eye_ref):
    i = pl.program_id(0)

    @pl.when(i == 0)
    def _build_eye():
        r = lax.broadcasted_iota(jnp.int32, (BT, BT), 0)
        c = lax.broadcasted_iota(jnp.int32, (BT, BT), 1)
        eye_ref[...] = (r == c).astype(jnp.float32)

    x = x_ref[...]
    w = w_ref[...]
    t = lax.dot_general(x, w, (((1,), (1,)), ((), ())),
                        preferred_element_type=jnp.float32)
    y_ref[...] = t
    iota_f = lax.broadcasted_iota(jnp.int32, (BT, O), 1).astype(jnp.float32)
    v1 = jnp.max(t, axis=1, keepdims=True)
    a1f = jnp.max(jnp.where(t == v1, iota_f, -1.0), axis=1, keepdims=True)
    t2 = jnp.where(iota_f == a1f, -jnp.inf, t)
    v2 = jnp.max(t2, axis=1, keepdims=True)
    a2f = jnp.max(jnp.where(t2 == v2, iota_f, -1.0), axis=1, keepdims=True)
    hi1 = jnp.floor(a1f * (1.0 / 32.0))
    lo1 = a1f - hi1 * 32.0
    hi2 = jnp.floor(a2f * (1.0 / 32.0))
    lo2 = a2f - hi2 * 32.0
    ab = jnp.concatenate([hi1, lo1, hi2, lo2], axis=1)
    abr = lax.dot_general(ab, eye_ref[...], (((0,), (0,)), ((), ())),
                          preferred_element_type=jnp.float32)
    rows = BT // CH
    a1_ref[pl.ds(i * rows, rows)] = (
        abr[0:1] * 32.0 + abr[1:2]).astype(jnp.int32).reshape(rows, CH)
    a2_ref[pl.ds(i * rows, rows)] = (
        abr[2:3] * 32.0 + abr[3:4]).astype(jnp.int32).reshape(rows, CH)


_matmul_top2 = pl.pallas_call(
    _matmul_top2_body,
    grid=(GRID,),
    in_specs=[
        pl.BlockSpec((BT, D), lambda i: (i, 0)),
        pl.BlockSpec((O, D), lambda i: (0, 0)),
    ],
    out_specs=[
        pl.BlockSpec((BT, O), lambda i: (i, 0)),
        pl.BlockSpec((B // CH, CH), lambda i: (0, 0)),
        pl.BlockSpec((B // CH, CH), lambda i: (0, 0)),
    ],
    out_shape=[
        jax.ShapeDtypeStruct((B, O), jnp.float32),
        jax.ShapeDtypeStruct((B // CH, CH), jnp.int32),
        jax.ShapeDtypeStruct((B // CH, CH), jnp.int32),
    ],
    scratch_shapes=[pltpu.VMEM((BT, BT), jnp.float32)],
)


def _scatter_body(x_hbm, a1_hbm, a2_hbm, out1_hbm, out2_hbm,
                  idx1_v, idx2_v, xrows_v, zero_v, acc1_sh, acc2_sh, xsems):
    cid = lax.axis_index("c")
    sid = lax.axis_index("s")
    wid = cid * 16 + sid
    base = wid * ROWS_PER_W

    xcopies = [
        pltpu.async_copy(x_hbm.at[pl.ds(base + j * CH, CH)],
                         xrows_v.at[pl.ds(j * CH, CH)], xsems.at[j])
        for j in range(NCH)
    ]

    pltpu.sync_copy(a1_hbm.at[pl.ds(wid * NCH, NCH)], idx1_v)
    pltpu.sync_copy(a2_hbm.at[pl.ds(wid * NCH, NCH)], idx2_v)

    def _zero_row(i, carry):
        for j in range(D // 16):
            zero_v[i, pl.ds(j * 16, 16)] = jnp.zeros((16,), jnp.float32)
        return carry
    lax.fori_loop(0, ZROWS, _zero_row, 0)

    pltpu.sync_copy(zero_v, acc1_sh.at[pl.ds(sid * ZROWS, ZROWS)])
    pltpu.sync_copy(zero_v, acc2_sh.at[pl.ds(sid * ZROWS, ZROWS)])

    plsc.subcore_barrier()

    for j in range(NCH):
        xcopies[j].wait()
        pltpu.sync_copy(xrows_v.at[pl.ds(j * CH, CH)],
                        acc1_sh.at[idx1_v.at[j]], add=True)
        pltpu.sync_copy(xrows_v.at[pl.ds(j * CH, CH)],
                        acc2_sh.at[idx2_v.at[j]], add=True)

    plsc.subcore_barrier()

    pltpu.sync_copy(acc1_sh.at[pl.ds(sid * ZROWS, ZROWS)],
                    out1_hbm.at[cid, pl.ds(sid * ZROWS, ZROWS)])
    pltpu.sync_copy(acc2_sh.at[pl.ds(sid * ZROWS, ZROWS)],
                    out2_hbm.at[cid, pl.ds(sid * ZROWS, ZROWS)])


@functools.cache
def _scatter_accumulate():
    return functools.partial(
        pl.kernel,
        out_type=[
            jax.ShapeDtypeStruct((2, O, D), jnp.float32),
            jax.ShapeDtypeStruct((2, O, D), jnp.float32),
        ],
        mesh=plsc.VectorSubcoreMesh(core_axis_name="c", subcore_axis_name="s"),
        scratch_types=[
            pltpu.VMEM((NCH, CH), jnp.int32),
            pltpu.VMEM((NCH, CH), jnp.int32),
            pltpu.VMEM((ROWS_PER_W, D), jnp.float32),
            pltpu.VMEM((ZROWS, D), jnp.float32),
            pltpu.VMEM_SHARED((O, D), jnp.float32),
            pltpu.VMEM_SHARED((O, D), jnp.float32),
            pltpu.SemaphoreType.DMA((NCH,)),
        ],
    )(_scatter_body)


def _finalize_body(w_ref, acc1_ref, acc2_ref, neww_ref):
    w = w_ref[...]
    s1 = acc1_ref[0] + acc1_ref[1]
    s2 = acc2_ref[0] + acc2_ref[1]
    dsb = s1 - ANTI * s2
    xx = jnp.sum(w * dsb, axis=1, keepdims=True)
    ds = dsb - xx * w
    nc = jnp.maximum(jnp.max(jnp.abs(ds)), PRECISION_FLOOR)
    neww_ref[...] = w + LR * (ds / nc)


_finalize = pl.pallas_call(
    _finalize_body,
    out_shape=jax.ShapeDtypeStruct((O, D), jnp.float32),
)


def kernel(input, W):
    y, a1, a2 = _matmul_top2(input, W)
    acc1, acc2 = _scatter_accumulate()(input, a1, a2)
    new_W = _finalize(W, acc1, acc2)
    return (y, new_W)
